# baseline (speedup 1.0000x reference)
import functools

import jax
import jax.numpy as jnp
from jax import lax
from jax.experimental import pallas as pl
from jax.experimental.pallas import tpu as pltpu
from jax.experimental.pallas import tpu_sc as plsc

B = 1024
L = 50
VOCAB = 100000
HIDDEN = 64

NC = 2
NS = 16
NW = NC * NS
H_PER_W = HIDDEN // NW
CHUNK = 128
NCHUNK = B // CHUNK
BG = CHUNK // 16


def _sc_body(words_hbm, w_hbm, b_hbm, out_hbm,
             wrow_v, wc0, wc1, outrow_v, bvec_v, semc0, semc1, semw):
    cid = lax.axis_index("c")
    sid = lax.axis_index("s")
    wid = sid * NC + cid
    wcs = [wc0, wc1]
    semcs = [semc0, semc1]

    pltpu.sync_copy(b_hbm, bvec_v.at[pl.ds(0, HIDDEN)])
    lanes = lax.iota(jnp.int32, 16)
    zeros16 = jnp.zeros((16,), jnp.float32)

    def words_copy(c, buf):
        return pltpu.make_async_copy(
            words_hbm.at[pl.ds(c * (CHUNK * L), CHUNK * L)],
            wcs[buf],
            semcs[buf],
        )

    def w_copy(h):
        return pltpu.make_async_copy(
            w_hbm.at[pl.ds(h * VOCAB, VOCAB)],
            wrow_v,
            semw,
        )

    w_copy(wid * H_PER_W).start()

    for hi in range(H_PER_W):
        h = wid * H_PER_W + hi
        w_copy(h).wait()
        bh = plsc.load_gather(bvec_v, [jnp.full((16,), 0, jnp.int32) + h])

        def chunk_body(c, _, bh=bh):
            pltpu.sync_copy(
                words_hbm.at[pl.ds(c * (CHUNK * L), CHUNK * L)], wc0
            )

            def bg_body(g, _, bh=bh, c=c):
                base = (g * 16 + lanes) * L
                acc0 = bh
                acc1 = zeros16
                for j in range(0, L, 2):
                    w0 = plsc.load_gather(wc0, [base + j])
                    w1 = plsc.load_gather(wc0, [base + (j + 1)])
                    acc0 = acc0 + plsc.load_gather(wrow_v, [w0])
                    acc1 = acc1 + plsc.load_gather(wrow_v, [w1])
                outrow_v[pl.ds(c * CHUNK + g * 16, 16)] = acc0 + acc1
                return 0

            lax.fori_loop(0, BG, bg_body, 0)
            return 0

        lax.fori_loop(0, NCHUNK, chunk_body, 0)

        if hi + 1 < H_PER_W:
            w_copy(h + 1).start()
        pltpu.sync_copy(outrow_v, out_hbm.at[h])


@functools.partial(jax.jit, donate_argnums=())
def _launch(words_flat, w_flat, b):
    mesh = plsc.VectorSubcoreMesh(core_axis_name="c", subcore_axis_name="s")
    f = pl.kernel(
        _sc_body,
        out_type=jax.ShapeDtypeStruct((HIDDEN, B), jnp.float32),
        mesh=mesh,
        scratch_types=[
            pltpu.VMEM((VOCAB,), jnp.float32),
            pltpu.VMEM((CHUNK * L,), jnp.int32),
            pltpu.VMEM((CHUNK * L,), jnp.int32),
            pltpu.VMEM((B,), jnp.float32),
            pltpu.VMEM((128,), jnp.float32),
            pltpu.SemaphoreType.DMA,
            pltpu.SemaphoreType.DMA,
            pltpu.SemaphoreType.DMA,
        ],
        compiler_params=pltpu.CompilerParams(needs_layout_passes=False),
    )
    return f(words_flat, w_flat, b)


def kernel(words, W, b):
    words_flat = words.reshape(-1).astype(jnp.int32)
    out_t = _launch(words_flat, W.reshape(-1), b)
    return out_t.T


# speedup vs baseline: 1.6232x; 1.6232x over previous
import functools

import jax
import jax.numpy as jnp
from jax import lax
from jax.experimental import pallas as pl
from jax.experimental.pallas import tpu as pltpu
from jax.experimental.pallas import tpu_sc as plsc

B = 1024
L = 50
VOCAB = 100000
HIDDEN = 64

NC = 2
NS = 16
NW = NC * NS
H_PER_W = HIDDEN // NW
CHUNK = 256
NCHUNK = B // CHUNK
BG = CHUNK // 16


def _sc_body(words_hbm, w_hbm, b_hbm, out_hbm, wrow_v, wc0, wc1, outrow_v, bvec_v):
    wid = lax.axis_index("s") * NC + lax.axis_index("c")
    pltpu.sync_copy(b_hbm, bvec_v.at[pl.ds(0, HIDDEN)])
    lanes = lax.iota(jnp.int32, 16)
    wcs = [wc0, wc1]

    def scoped(sem0, sem1):
        sems = [sem0, sem1]

        def words_copy(c):
            return pltpu.async_copy(
                words_hbm.at[pl.ds(c * (CHUNK * L), CHUNK * L)],
                wcs[c % 2],
                sems[c % 2],
            )

        for hi in range(H_PER_W):
            h = wid * H_PER_W + hi
            handles = [words_copy(0), words_copy(1)]
            pltpu.sync_copy(w_hbm.at[h], wrow_v)
            bh = plsc.load_gather(bvec_v, [jnp.full((16,), 0, jnp.int32) + h])
            for c in range(NCHUNK):
                handles[c % 2].wait()
                wordsc = wcs[c % 2]

                def bg_body(g, _, bh=bh, wordsc=wordsc):
                    base = (g * 16 + lanes) * L
                    acc0 = bh
                    acc1 = jnp.zeros((16,), jnp.float32)
                    for j in range(0, L, 2):
                        w0 = plsc.load_gather(wordsc, [base + j])
                        acc0 = acc0 + plsc.load_gather(wrow_v, [w0])
                        w1 = plsc.load_gather(wordsc, [base + (j + 1)])
                        acc1 = acc1 + plsc.load_gather(wrow_v, [w1])
                    outrow_v[pl.ds(g * 16, 16)] = acc0 + acc1
                    return 0

                lax.fori_loop(0, BG, bg_body, 0)
                if c + 2 < NCHUNK:
                    handles[c % 2] = words_copy(c + 2)
                pltpu.sync_copy(outrow_v, out_hbm.at[h, pl.ds(c * CHUNK, CHUNK)])

    pl.run_scoped(scoped, pltpu.SemaphoreType.DMA, pltpu.SemaphoreType.DMA)


@functools.partial(jax.jit, donate_argnums=())
def _launch(words_flat, W, b):
    mesh = plsc.VectorSubcoreMesh(core_axis_name="c", subcore_axis_name="s")
    f = pl.kernel(
        _sc_body,
        out_type=jax.ShapeDtypeStruct((HIDDEN, B), jnp.float32),
        mesh=mesh,
        scratch_types=[
            pltpu.VMEM((VOCAB,), jnp.float32),
            pltpu.VMEM((CHUNK * L,), jnp.int32),
            pltpu.VMEM((CHUNK * L,), jnp.int32),
            pltpu.VMEM((CHUNK,), jnp.float32),
            pltpu.VMEM((128,), jnp.float32),
        ],
        compiler_params=pltpu.CompilerParams(needs_layout_passes=False),
    )
    return f(words_flat, W, b)


def kernel(words, W, b):
    words_flat = words.reshape(-1).astype(jnp.int32)
    out_t = _launch(words_flat, W, b)
    return out_t.T


# speedup vs baseline: 1.7108x; 1.0539x over previous
import functools

import jax
import jax.numpy as jnp
from jax import lax
from jax.experimental import pallas as pl
from jax.experimental.pallas import tpu as pltpu
from jax.experimental.pallas import tpu_sc as plsc

B = 1024
L = 50
VOCAB = 100000
HIDDEN = 64

NC = 2
NS = 16
NW = NC * NS
H_PER_W = HIDDEN // NW
CHUNK = 256
NCHUNK = B // CHUNK
BG = CHUNK // 16


def _sc_body(words_hbm, w_hbm, b_hbm, out_hbm, wrow_v, wc0, wc1, outrow_v, bvec_v):
    wid = lax.axis_index("s") * NC + lax.axis_index("c")
    pltpu.sync_copy(b_hbm, bvec_v.at[pl.ds(0, HIDDEN)])
    lanes = lax.iota(jnp.int32, 16)
    wcs = [wc0, wc1]

    def scoped(sem0, sem1, semw):
        sems = [sem0, sem1]

        def words_copy(c):
            return pltpu.async_copy(
                words_hbm.at[pl.ds(c * (CHUNK * L), CHUNK * L)],
                wcs[c % 2],
                sems[c % 2],
            )

        def w_copy(h):
            return pltpu.async_copy(w_hbm.at[h], wrow_v, semw)

        wh = w_copy(wid * H_PER_W)
        for hi in range(H_PER_W):
            h = wid * H_PER_W + hi
            handles = [words_copy(0), words_copy(1)]
            wh.wait()
            bh = plsc.load_gather(bvec_v, [jnp.full((16,), 0, jnp.int32) + h])
            for c in range(NCHUNK):
                handles[c % 2].wait()
                wordsc = wcs[c % 2]

                def bg_body(g, _, bh=bh, wordsc=wordsc, c=c):
                    base = (g * 16 + lanes) * L
                    acc0 = bh
                    acc1 = jnp.zeros((16,), jnp.float32)
                    acc2 = jnp.zeros((16,), jnp.float32)
                    acc3 = jnp.zeros((16,), jnp.float32)
                    for j in range(0, L, 4):
                        w0 = plsc.load_gather(wordsc, [base + j])
                        acc0 = acc0 + plsc.load_gather(wrow_v, [w0])
                        w1 = plsc.load_gather(wordsc, [base + (j + 1)])
                        acc1 = acc1 + plsc.load_gather(wrow_v, [w1])
                        if j + 2 < L:
                            w2 = plsc.load_gather(wordsc, [base + (j + 2)])
                            acc2 = acc2 + plsc.load_gather(wrow_v, [w2])
                            w3 = plsc.load_gather(wordsc, [base + (j + 3)])
                            acc3 = acc3 + plsc.load_gather(wrow_v, [w3])
                    outrow_v[pl.ds(c * CHUNK + g * 16, 16)] = (
                        (acc0 + acc1) + (acc2 + acc3))
                    return 0

                lax.fori_loop(0, BG, bg_body, 0)
                if hi + 1 < H_PER_W and c == NCHUNK - 1:
                    wh = w_copy(h + 1)
                if c + 2 < NCHUNK:
                    handles[c % 2] = words_copy(c + 2)
            pltpu.sync_copy(outrow_v, out_hbm.at[h])

    pl.run_scoped(scoped, pltpu.SemaphoreType.DMA, pltpu.SemaphoreType.DMA,
                  pltpu.SemaphoreType.DMA)


@functools.partial(jax.jit, donate_argnums=())
def _launch(words_flat, W, b):
    mesh = plsc.VectorSubcoreMesh(core_axis_name="c", subcore_axis_name="s")
    f = pl.kernel(
        _sc_body,
        out_type=jax.ShapeDtypeStruct((HIDDEN, B), jnp.float32),
        mesh=mesh,
        scratch_types=[
            pltpu.VMEM((VOCAB,), jnp.float32),
            pltpu.VMEM((CHUNK * L,), jnp.int32),
            pltpu.VMEM((CHUNK * L,), jnp.int32),
            pltpu.VMEM((B,), jnp.float32),
            pltpu.VMEM((128,), jnp.float32),
        ],
        compiler_params=pltpu.CompilerParams(needs_layout_passes=False),
    )
    return f(words_flat, W, b)


def kernel(words, W, b):
    words_flat = words.reshape(-1).astype(jnp.int32)
    out_t = _launch(words_flat, W, b)
    return out_t.T
